# baseline (speedup 1.0000x reference)
import functools

import jax
import jax.numpy as jnp
from jax import lax
from jax.experimental import pallas as pl
from jax.experimental.pallas import tpu as pltpu
from jax.experimental.pallas import tpu_sc as plsc

N, DEG, D = 10000, 32, 128
G4 = 4 * D
NP = 10240


@functools.cache
def _make_sc_gather(n_edges, d):
    info = plsc.get_sparse_core_info()
    nw = info.num_cores * info.num_subcores
    per_w = n_edges // nw
    assert per_w * nw == n_edges
    chunk = 80
    nbuf = 4
    n_ch = per_w // chunk
    n_outer = n_ch // nbuf
    assert n_ch * chunk == per_w and n_outer * nbuf == n_ch
    mesh = plsc.VectorSubcoreMesh(core_axis_name="c", subcore_axis_name="s")

    @functools.partial(
        pl.kernel,
        mesh=mesh,
        out_type=jax.ShapeDtypeStruct((n_edges, d), jnp.float32),
        scratch_types=[
            pltpu.VMEM((per_w,), jnp.int32),
            pltpu.VMEM((nbuf, chunk, d), jnp.float32),
        ]
        + [pltpu.SemaphoreType.DMA] * (2 * nbuf),
    )
    def gather(table_hbm, idx_hbm, out_hbm, idx_v, rows_v, *sems):
        gsems, osems = sems[:nbuf], sems[nbuf:]
        wid = lax.axis_index("s") * info.num_cores + lax.axis_index("c")
        base = wid * per_w
        pltpu.sync_copy(idx_hbm.at[pl.ds(base, per_w)], idx_v)

        def g_start(c, b):
            return pltpu.async_copy(
                table_hbm.at[idx_v.at[pl.ds(c * chunk, chunk)]],
                rows_v.at[b], gsems[b])

        def o_start(c, b):
            return pltpu.async_copy(
                rows_v.at[b], out_hbm.at[pl.ds(base + c * chunk, chunk)],
                osems[b])

        def o_wait(c, b):
            pltpu.make_async_copy(
                rows_v.at[b], out_hbm.at[pl.ds(base + c * chunk, chunk)],
                osems[b]).wait()

        hs = [g_start(b, b) for b in range(nbuf)]
        for b in range(nbuf):
            hs[b].wait()
            o_start(b, b)

        def outer(i, t):
            c0 = i * nbuf
            hs = []
            for b in range(nbuf):
                o_wait(c0 - nbuf + b, b)
                hs.append(g_start(c0 + b, b))
            for b in range(nbuf):
                hs[b].wait()
                o_start(c0 + b, b)
            return t

        lax.fori_loop(1, n_outer, outer, 0)
        for b in range(nbuf):
            o_wait((n_outer - 1) * nbuf + b, b)

    return gather


def _sc_gather(table, idx):
    return _make_sc_gather(NP * DEG, D)(table, idx)


def _layer(feat, msgs, wihT, whhT, bias, wselfT, wneighT, b, act):
    B = 512
    grid = NP // B

    Bh = B // 2

    def _sig(v):
        return 0.5 * jnp.tanh(0.5 * v) + 0.5

    def body(feat_ref, msgs_ref, wihT_ref, whhT_ref, bias_ref, wselfT_ref,
             wneighT_ref, b_ref, out_ref, ip_ref):
        m2 = msgs_ref[...].reshape(DEG * B, D).astype(jnp.bfloat16)
        ip_ref[...] = (
            jnp.dot(m2, wihT_ref[...], preferred_element_type=jnp.float32)
            + bias_ref[...]
        )
        whhT_v = whhT_ref[...]

        def step(t, hc):
            h, c = hc
            gates = ip_ref[pl.ds(t * B, B), :] + jnp.dot(
                h.astype(jnp.bfloat16), whhT_v,
                preferred_element_type=jnp.float32)
            i = jax.nn.sigmoid(gates[:, 0:D])
            f = jax.nn.sigmoid(gates[:, D:2 * D])
            g = jnp.tanh(gates[:, 2 * D:3 * D])
            o = jax.nn.sigmoid(gates[:, 3 * D:4 * D])
            c = f * c + i * g
            h = o * jnp.tanh(c)
            return (h, c)

        z = jnp.zeros((B, D), jnp.float32)
        h, _ = lax.fori_loop(0, DEG, step, (z, z))
        r = (
            jnp.dot(feat_ref[...], wselfT_ref[...],
                    preferred_element_type=jnp.float32)
            + jnp.dot(h, wneighT_ref[...], preferred_element_type=jnp.float32)
            + b_ref[...]
        )
        out_ref[...] = act(r)

    return pl.pallas_call(
        body,
        grid=(grid,),
        in_specs=[
            pl.BlockSpec((B, D), lambda i: (i, 0)),
            pl.BlockSpec((DEG, B, D), lambda i: (0, i, 0)),
            pl.BlockSpec((D, G4), lambda i: (0, 0)),
            pl.BlockSpec((D, G4), lambda i: (0, 0)),
            pl.BlockSpec((1, G4), lambda i: (0, 0)),
            pl.BlockSpec((D, D), lambda i: (0, 0)),
            pl.BlockSpec((D, D), lambda i: (0, 0)),
            pl.BlockSpec((1, D), lambda i: (0, 0)),
        ],
        out_specs=pl.BlockSpec((B, D), lambda i: (i, 0)),
        out_shape=jax.ShapeDtypeStruct((NP, D), jnp.float32),
        scratch_shapes=[pltpu.VMEM((DEG * B, G4), jnp.float32)],
    )(feat, msgs, wihT.astype(jnp.bfloat16), whhT.astype(jnp.bfloat16),
      bias.reshape(1, G4), wselfT, wneighT, b.reshape(1, D))


def kernel(x, edge_index, Wih1, Whh1, bih1, bhh1, Wself1, Wneigh1, b1,
           Wih2, Whh2, bih2, bhh2, Wself2, Wneigh2, b2):
    src_t = jnp.pad(edge_index[0].reshape(N, DEG).T, ((0, 0), (0, NP - N))
                    ).reshape(NP * DEG)
    xp = jnp.pad(x, ((0, NP - N), (0, 0)))
    msgs1 = _sc_gather(xp, src_t).reshape(DEG, NP, D)
    h1 = _layer(xp, msgs1, Wih1.T, Whh1.T, bih1 + bhh1, Wself1.T, Wneigh1.T,
                b1, jax.nn.relu)
    msgs2 = _sc_gather(h1, src_t).reshape(DEG, NP, D)
    out = _layer(h1, msgs2, Wih2.T, Whh2.T, bih2 + bhh2, Wself2.T,
                 Wneigh2.T, b2, jax.nn.sigmoid)
    return out[:N, :]


# speedup vs baseline: 1.5082x; 1.5082x over previous
import functools

import jax
import jax.numpy as jnp
from jax import lax
from jax.experimental import pallas as pl
from jax.experimental.pallas import tpu as pltpu
from jax.experimental.pallas import tpu_sc as plsc

N, DEG, D = 10000, 32, 128
G4 = 4 * D
NP = 10240


@functools.cache
def _make_sc_gather(d):
    info = plsc.get_sparse_core_info()
    nw = info.num_cores * info.num_subcores
    assert nw == DEG
    per_w = N
    chunk = 80
    nbuf = 5
    n_ch = per_w // chunk
    n_outer = n_ch // nbuf
    assert n_ch * chunk == per_w and n_outer * nbuf == n_ch
    mesh = plsc.VectorSubcoreMesh(core_axis_name="c", subcore_axis_name="s")

    @functools.partial(
        pl.kernel,
        mesh=mesh,
        out_type=jax.ShapeDtypeStruct((DEG * NP, d), jnp.float32),
        scratch_types=[
            pltpu.VMEM((per_w,), jnp.int32),
            pltpu.VMEM((nbuf, chunk, d), jnp.float32),
        ]
        + [pltpu.SemaphoreType.DMA] * (2 * nbuf),
    )
    def gather(table_hbm, idx_hbm, out_hbm, idx_v, rows_v, *sems):
        gsems, osems = sems[:nbuf], sems[nbuf:]
        wid = lax.axis_index("s") * info.num_cores + lax.axis_index("c")
        obase = wid * NP
        pltpu.sync_copy(idx_hbm.at[pl.ds(wid * per_w, per_w)], idx_v)

        def g_start(c, b):
            return pltpu.async_copy(
                table_hbm.at[idx_v.at[pl.ds(c * chunk, chunk)]],
                rows_v.at[b], gsems[b])

        def o_start(c, b):
            return pltpu.async_copy(
                rows_v.at[b], out_hbm.at[pl.ds(obase + c * chunk, chunk)],
                osems[b])

        def o_wait(c, b):
            pltpu.make_async_copy(
                rows_v.at[b], out_hbm.at[pl.ds(obase + c * chunk, chunk)],
                osems[b]).wait()

        hs = [g_start(b, b) for b in range(nbuf)]
        for b in range(nbuf):
            hs[b].wait()
            o_start(b, b)

        def outer(i, t):
            c0 = i * nbuf
            hs = []
            for b in range(nbuf):
                o_wait(c0 - nbuf + b, b)
                hs.append(g_start(c0 + b, b))
            for b in range(nbuf):
                hs[b].wait()
                o_start(c0 + b, b)
            return t

        lax.fori_loop(1, n_outer, outer, 0)
        for b in range(nbuf):
            o_wait((n_outer - 1) * nbuf + b, b)

    return gather


def _sc_gather(table, idx):
    return _make_sc_gather(D)(table, idx)


def _layer(feat, msgs, wihT, whhT, bias, wselfT, wneighT, b, act):
    B = 512
    grid = NP // B

    Bh = B // 2

    def _sig(v):
        return 0.5 * jnp.tanh(0.5 * v) + 0.5

    def body(feat_ref, msgs_ref, wihT_ref, whhT_ref, bias_ref, wselfT_ref,
             wneighT_ref, b_ref, out_ref, ip_ref):
        m2 = msgs_ref[...].reshape(DEG * B, D).astype(jnp.bfloat16)
        ip_ref[...] = (
            jnp.dot(m2, wihT_ref[...], preferred_element_type=jnp.float32)
            + bias_ref[...]
        )
        whhT_v = whhT_ref[...]

        def step(t, hc):
            h, c = hc
            gates = ip_ref[pl.ds(t * B, B), :] + jnp.dot(
                h.astype(jnp.bfloat16), whhT_v,
                preferred_element_type=jnp.float32)
            i = jax.nn.sigmoid(gates[:, 0:D])
            f = jax.nn.sigmoid(gates[:, D:2 * D])
            g = jnp.tanh(gates[:, 2 * D:3 * D])
            o = jax.nn.sigmoid(gates[:, 3 * D:4 * D])
            c = f * c + i * g
            h = o * jnp.tanh(c)
            return (h, c)

        z = jnp.zeros((B, D), jnp.float32)
        h, _ = lax.fori_loop(0, DEG, step, (z, z))
        r = (
            jnp.dot(feat_ref[...], wselfT_ref[...],
                    preferred_element_type=jnp.float32)
            + jnp.dot(h, wneighT_ref[...], preferred_element_type=jnp.float32)
            + b_ref[...]
        )
        out_ref[...] = act(r)

    return pl.pallas_call(
        body,
        grid=(grid,),
        in_specs=[
            pl.BlockSpec((B, D), lambda i: (i, 0)),
            pl.BlockSpec((DEG, B, D), lambda i: (0, i, 0)),
            pl.BlockSpec((D, G4), lambda i: (0, 0)),
            pl.BlockSpec((D, G4), lambda i: (0, 0)),
            pl.BlockSpec((1, G4), lambda i: (0, 0)),
            pl.BlockSpec((D, D), lambda i: (0, 0)),
            pl.BlockSpec((D, D), lambda i: (0, 0)),
            pl.BlockSpec((1, D), lambda i: (0, 0)),
        ],
        out_specs=pl.BlockSpec((B, D), lambda i: (i, 0)),
        out_shape=jax.ShapeDtypeStruct((NP, D), jnp.float32),
        scratch_shapes=[pltpu.VMEM((DEG * B, G4), jnp.float32)],
    )(feat, msgs, wihT.astype(jnp.bfloat16), whhT.astype(jnp.bfloat16),
      bias.reshape(1, G4), wselfT, wneighT, b.reshape(1, D))


def kernel(x, edge_index, Wih1, Whh1, bih1, bhh1, Wself1, Wneigh1, b1,
           Wih2, Whh2, bih2, bhh2, Wself2, Wneigh2, b2):
    src_t = edge_index[0].reshape(N, DEG).T.reshape(N * DEG)
    xp = jnp.pad(x, ((0, NP - N), (0, 0)))
    msgs1 = _sc_gather(xp, src_t).reshape(DEG, NP, D)
    h1 = _layer(xp, msgs1, Wih1.T, Whh1.T, bih1 + bhh1, Wself1.T, Wneigh1.T,
                b1, jax.nn.relu)
    msgs2 = _sc_gather(h1, src_t).reshape(DEG, NP, D)
    out = _layer(h1, msgs2, Wih2.T, Whh2.T, bih2 + bhh2, Wself2.T,
                 Wneigh2.T, b2, jax.nn.sigmoid)
    return out[:N, :]


# speedup vs baseline: 1.6020x; 1.0622x over previous
import functools

import jax
import jax.numpy as jnp
from jax import lax
from jax.experimental import pallas as pl
from jax.experimental.pallas import tpu as pltpu
from jax.experimental.pallas import tpu_sc as plsc

N, DEG, D = 10000, 32, 128
G4 = 4 * D
NP = 10240


@functools.cache
def _make_sc_gather(d):
    info = plsc.get_sparse_core_info()
    nw = info.num_cores * info.num_subcores
    assert nw == DEG
    per_w = N
    chunk = 80
    nbuf = 5
    n_ch = per_w // chunk
    n_outer = n_ch // nbuf
    assert n_ch * chunk == per_w and n_outer * nbuf == n_ch
    mesh = plsc.VectorSubcoreMesh(core_axis_name="c", subcore_axis_name="s")

    @functools.partial(
        pl.kernel,
        mesh=mesh,
        out_type=jax.ShapeDtypeStruct((DEG * NP, d), jnp.float32),
        scratch_types=[
            pltpu.VMEM((per_w,), jnp.int32),
            pltpu.VMEM((nbuf, chunk, d), jnp.float32),
        ]
        + [pltpu.SemaphoreType.DMA] * (2 * nbuf),
    )
    def gather(table_hbm, idx_hbm, out_hbm, idx_v, rows_v, *sems):
        gsems, osems = sems[:nbuf], sems[nbuf:]
        wid = lax.axis_index("s") * info.num_cores + lax.axis_index("c")
        obase = wid * NP
        pltpu.sync_copy(idx_hbm.at[pl.ds(wid * per_w, per_w)], idx_v)

        def g_start(c, b):
            return pltpu.async_copy(
                table_hbm.at[idx_v.at[pl.ds(c * chunk, chunk)]],
                rows_v.at[b], gsems[b])

        def o_start(c, b):
            return pltpu.async_copy(
                rows_v.at[b], out_hbm.at[pl.ds(obase + c * chunk, chunk)],
                osems[b])

        def o_wait(c, b):
            pltpu.make_async_copy(
                rows_v.at[b], out_hbm.at[pl.ds(obase + c * chunk, chunk)],
                osems[b]).wait()

        hs = [g_start(b, b) for b in range(nbuf)]
        for b in range(nbuf):
            hs[b].wait()
            o_start(b, b)

        def outer(i, t):
            c0 = i * nbuf
            hs = []
            for b in range(nbuf):
                o_wait(c0 - nbuf + b, b)
                hs.append(g_start(c0 + b, b))
            for b in range(nbuf):
                hs[b].wait()
                o_start(c0 + b, b)
            return t

        lax.fori_loop(1, n_outer, outer, 0)
        for b in range(nbuf):
            o_wait((n_outer - 1) * nbuf + b, b)

    return gather


def _sc_gather(table, idx):
    return _make_sc_gather(D)(table, idx)


def _layer(feat, msgs, wihT, whhT, bias, wselfT, wneighT, b, act):
    B = 512
    grid = NP // B

    Bh = B // 2

    def _sig(v):
        return 0.5 * jnp.tanh(0.5 * v) + 0.5

    def body(feat_ref, msgs_ref, wihT_ref, whhT_ref, bias_ref, wselfT_ref,
             wneighT_ref, b_ref, out_ref, ip_ref):
        m2 = msgs_ref[...].reshape(DEG * B, D).astype(jnp.bfloat16)
        ip_ref[...] = (
            jnp.dot(m2, wihT_ref[...], preferred_element_type=jnp.float32)
            + bias_ref[...]
        ).astype(jnp.bfloat16)
        whhT_v = whhT_ref[...]

        def step(t, hc):
            new = []
            for k in range(2):
                h, c = hc[k]
                gates = ip_ref[pl.ds(t * B + k * Bh, Bh), :].astype(
                    jnp.float32) + jnp.dot(
                    h.astype(jnp.bfloat16), whhT_v,
                    preferred_element_type=jnp.float32)
                i = _sig(gates[:, 0:D])
                f = _sig(gates[:, D:2 * D])
                g = jnp.tanh(gates[:, 2 * D:3 * D])
                o = _sig(gates[:, 3 * D:4 * D])
                c = f * c + i * g
                h = o * jnp.tanh(c)
                new.append((h, c))
            return tuple(new)

        z = jnp.zeros((Bh, D), jnp.float32)
        hcs = lax.fori_loop(0, DEG, step, ((z, z), (z, z)))
        for k in range(2):
            sl = pl.ds(k * Bh, Bh)
            r = (
                jnp.dot(feat_ref[sl, :], wselfT_ref[...],
                        preferred_element_type=jnp.float32)
                + jnp.dot(hcs[k][0], wneighT_ref[...],
                          preferred_element_type=jnp.float32)
                + b_ref[...]
            )
            out_ref[sl, :] = act(r)

    return pl.pallas_call(
        body,
        grid=(grid,),
        in_specs=[
            pl.BlockSpec((B, D), lambda i: (i, 0)),
            pl.BlockSpec((DEG, B, D), lambda i: (0, i, 0)),
            pl.BlockSpec((D, G4), lambda i: (0, 0)),
            pl.BlockSpec((D, G4), lambda i: (0, 0)),
            pl.BlockSpec((1, G4), lambda i: (0, 0)),
            pl.BlockSpec((D, D), lambda i: (0, 0)),
            pl.BlockSpec((D, D), lambda i: (0, 0)),
            pl.BlockSpec((1, D), lambda i: (0, 0)),
        ],
        out_specs=pl.BlockSpec((B, D), lambda i: (i, 0)),
        out_shape=jax.ShapeDtypeStruct((NP, D), jnp.float32),
        scratch_shapes=[pltpu.VMEM((DEG * B, G4), jnp.bfloat16)],
    )(feat, msgs, wihT.astype(jnp.bfloat16), whhT.astype(jnp.bfloat16),
      bias.reshape(1, G4), wselfT, wneighT, b.reshape(1, D))


def kernel(x, edge_index, Wih1, Whh1, bih1, bhh1, Wself1, Wneigh1, b1,
           Wih2, Whh2, bih2, bhh2, Wself2, Wneigh2, b2):
    src_t = edge_index[0].reshape(N, DEG).T.reshape(N * DEG)
    xp = jnp.pad(x, ((0, NP - N), (0, 0)))
    msgs1 = _sc_gather(xp, src_t).reshape(DEG, NP, D)
    h1 = _layer(xp, msgs1, Wih1.T, Whh1.T, bih1 + bhh1, Wself1.T, Wneigh1.T,
                b1, jax.nn.relu)
    msgs2 = _sc_gather(h1, src_t).reshape(DEG, NP, D)
    out = _layer(h1, msgs2, Wih2.T, Whh2.T, bih2 + bhh2, Wself2.T,
                 Wneigh2.T, b2, jax.nn.sigmoid)
    return out[:N, :]
